# baseline (speedup 1.0000x reference)
import functools

import jax
import jax.numpy as jnp
from jax import lax
from jax.experimental import pallas as pl
from jax.experimental.pallas import tpu as pltpu
from jax.experimental.pallas import tpu_sc as plsc

D = 256
L = 16
DC2 = D // 32
NC = 2
NS = 16
NW = NC * NS
CE = 16
WCHUNKS = 320
SB = 8


def _make_sc_kernel(e_pad: int, n_nodes: int):
    assert e_pad == NW * WCHUNKS * CE
    mesh = plsc.VectorSubcoreMesh(core_axis_name="c", subcore_axis_name="s")
    n_blocks = n_nodes // SB
    blocks_main = n_blocks // NS
    blocks_extra = n_blocks - blocks_main * NS

    @functools.partial(
        pl.kernel,
        mesh=mesh,
        out_type=jax.ShapeDtypeStruct((e_pad,), jnp.float32),
        compiler_params=pltpu.CompilerParams(needs_layout_passes=False),
        scratch_types=[
            pltpu.VMEM((WCHUNKS // 2, 2 * CE), jnp.int32),
            pltpu.VMEM((2 * CE, D // 2), jnp.int32),
            pltpu.VMEM((2 * CE, D // 2), jnp.int32),
            pltpu.VMEM((WCHUNKS * CE // 2,), jnp.float32),
            pltpu.VMEM((D,), jnp.float32),
            pltpu.VMEM((L,), jnp.float32),
            pltpu.VMEM((SB, D), jnp.float32),
            pltpu.VMEM_SHARED((10000, D // 2), jnp.int32),
            pltpu.SemaphoreType.DMA,
            pltpu.SemaphoreType.DMA,
        ],
    )
    def sc_kernel(h_hbm, uvidx_hbm, w_hbm, b_hbm, out_hbm,
                  uvidx_v, uv_buf0, uv_buf1, out_v, w_v, b_v, stage_v,
                  h_sp, sem0, sem1):
        cidx = lax.axis_index("c")
        sidx = lax.axis_index("s")

        def stage_block(b, carry):
            row0 = b * SB
            pltpu.sync_copy(h_hbm.at[pl.ds(row0, SB)], stage_v)
            for r in range(SB):
                for c in range(DC2):
                    a0 = stage_v[r, pl.ds(c * 32, L)]
                    a1 = stage_v[r, pl.ds(c * 32 + L, L)]
                    packed = plsc.pack(
                        a0, a1, format=plsc.PackFormat.INTERLEAVED)
                    uv_buf0[r, pl.ds(c * L, L)] = plsc.bitcast(
                        packed, jnp.int32)
            pltpu.sync_copy(uv_buf0.at[pl.ds(0, SB)],
                            h_sp.at[pl.ds(row0, SB)])
            return carry

        tile_blocks = blocks_main + (sidx >= NS - blocks_extra)
        b0 = sidx * blocks_main + jnp.maximum(
            sidx - (NS - blocks_extra), 0) if blocks_extra else (
                sidx * blocks_main)
        lax.fori_loop(b0, b0 + tile_blocks, stage_block, 0)
        pltpu.sync_copy(w_hbm, w_v)
        pltpu.sync_copy(b_hbm, b_v)
        plsc.subcore_barrier()

        w_regs = [w_v[pl.ds(j * L, L)] for j in range(2 * DC2)]
        b_reg = b_v[...]
        iota = lax.iota(jnp.int32, L)
        lane_masks = [iota == e for e in range(L)]
        bufs = (uv_buf0, uv_buf1)
        sems = (sem0, sem1)

        wid = sidx * NC + cidx
        HC = WCHUNKS // 2

        def start(k, slot):
            pltpu.async_copy(h_sp.at[uvidx_v.at[k]], bufs[slot], sems[slot])

        def wait(k, slot):
            pltpu.make_async_copy(
                h_sp.at[uvidx_v.at[k]], bufs[slot], sems[slot]).wait()

        def compute(k, slot):
            buf = bufs[slot]
            accs = [None] * L
            for c in range(DC2):
                w0 = w_regs[2 * c]
                w1 = w_regs[2 * c + 1]
                for e in range(L):
                    u32 = plsc.bitcast(buf[e, pl.ds(c * L, L)], jnp.bfloat16)
                    v32 = plsc.bitcast(buf[L + e, pl.ds(c * L, L)],
                                       jnp.bfloat16)
                    p32 = u32 * v32
                    p0, p1 = plsc.unpack(
                        p32, format=plsc.PackFormat.INTERLEAVED)
                    p = p0 * w0 + p1 * w1
                    accs[e] = p if c == 0 else accs[e] + p
            tot = b_reg
            for e in range(L):
                s = jnp.sum(accs[e])
                tot = jnp.where(lane_masks[e],
                                jnp.broadcast_to(s, (L,)), tot)
            out_v[pl.ds(k * CE, L)] = tot + b_reg

        npair = HC // 2

        def half_body(hh, carry):
            half_base = wid * WCHUNKS + hh * HC
            pltpu.sync_copy(uvidx_hbm.at[pl.ds(half_base, HC)], uvidx_v)
            start(0, 0)

            def pair_body(p, carry2):
                k0 = p * 2
                wait(k0, 0)
                start(k0 + 1, 1)
                compute(k0, 0)
                wait(k0 + 1, 1)
                start(jnp.minimum(k0 + 2, HC - 1), 0)
                compute(k0 + 1, 1)
                return carry2

            lax.fori_loop(0, npair, pair_body, 0)
            wait(HC - 1, 0)
            pltpu.sync_copy(out_v, out_hbm.at[pl.ds(half_base * CE, HC * CE)])
            return carry

        lax.fori_loop(0, 2, half_body, 0)

    return sc_kernel


def kernel(h, edge_index, W1_w, W1_b):
    e = edge_index.shape[1]
    e_pad = NW * WCHUNKS * CE
    assert e <= e_pad
    src = edge_index[0].astype(jnp.int32)
    dst = edge_index[1].astype(jnp.int32)
    pad = e_pad - e
    if pad:
        src = jnp.concatenate([src, jnp.zeros((pad,), jnp.int32)])
        dst = jnp.concatenate([dst, jnp.zeros((pad,), jnp.int32)])
    uvidx = jnp.concatenate(
        [src.reshape(-1, CE), dst.reshape(-1, CE)], axis=1)
    w = W1_w.reshape(D).astype(jnp.float32)
    bvec = jnp.broadcast_to(W1_b.reshape(1).astype(jnp.float32), (L,))
    n = h.shape[0]
    out = _make_sc_kernel(e_pad, n)(
        h.astype(jnp.float32), uvidx, w, bvec)
    return out[:e]


# speedup vs baseline: 1.2053x; 1.2053x over previous
import functools

import jax
import jax.numpy as jnp
from jax import lax
from jax.experimental import pallas as pl
from jax.experimental.pallas import tpu as pltpu
from jax.experimental.pallas import tpu_sc as plsc

D = 256
L = 16
DC2 = D // 32
NC = 2
NS = 16
NW = NC * NS
CE = 32
WCHUNKS = 160
SB = 8


def _make_sc_kernel(e_pad: int, n_nodes: int):
    assert e_pad == NW * WCHUNKS * CE
    mesh = plsc.VectorSubcoreMesh(core_axis_name="c", subcore_axis_name="s")
    n_blocks = n_nodes // SB
    blocks_main = n_blocks // NS
    blocks_extra = n_blocks - blocks_main * NS

    @functools.partial(
        pl.kernel,
        mesh=mesh,
        out_type=jax.ShapeDtypeStruct((e_pad,), jnp.float32),
        compiler_params=pltpu.CompilerParams(needs_layout_passes=False),
        scratch_types=[
            pltpu.VMEM((WCHUNKS // 2, 2 * CE), jnp.int32),
            pltpu.VMEM((2 * CE, D // 2), jnp.int32),
            pltpu.VMEM((2 * CE, D // 2), jnp.int32),
            pltpu.VMEM((WCHUNKS * CE // 2,), jnp.float32),
            pltpu.VMEM((D,), jnp.float32),
            pltpu.VMEM((L,), jnp.float32),
            pltpu.VMEM((SB, D), jnp.float32),
            pltpu.VMEM((SB, D), jnp.float32),
            pltpu.VMEM_SHARED((10000, D // 2), jnp.int32),
            pltpu.SemaphoreType.DMA,
            pltpu.SemaphoreType.DMA,
        ],
    )
    def sc_kernel(h_hbm, uvidx_hbm, w_hbm, b_hbm, out_hbm,
                  uvidx_v, uv_buf0, uv_buf1, out_v, w_v, b_v,
                  stage_v0, stage_v1, h_sp, sem0, sem1):
        cidx = lax.axis_index("c")
        sidx = lax.axis_index("s")
        sems = (sem0, sem1)

        stage_bufs = (stage_v0, stage_v1)

        def stage_start(b, slot):
            pltpu.async_copy(h_hbm.at[pl.ds(b * SB, SB)],
                             stage_bufs[slot], sems[slot])

        def stage_wait(b, slot):
            pltpu.make_async_copy(h_hbm.at[pl.ds(b * SB, SB)],
                                  stage_bufs[slot], sems[slot]).wait()

        def stage_pack(b, slot):
            sv = stage_bufs[slot]
            for r in range(SB):
                for c in range(DC2):
                    a0 = sv[r, pl.ds(c * 32, L)]
                    a1 = sv[r, pl.ds(c * 32 + L, L)]
                    packed = plsc.pack(
                        a0, a1, format=plsc.PackFormat.INTERLEAVED)
                    uv_buf0[r, pl.ds(c * L, L)] = plsc.bitcast(
                        packed, jnp.int32)
            pltpu.sync_copy(uv_buf0.at[pl.ds(0, SB)],
                            h_sp.at[pl.ds(b * SB, SB)])

        tile_blocks = blocks_main + (sidx >= NS - blocks_extra)
        b0 = sidx * blocks_main + jnp.maximum(
            sidx - (NS - blocks_extra), 0) if blocks_extra else (
                sidx * blocks_main)
        b_end = b0 + tile_blocks
        stage_start(b0, 0)

        def stage_pair(p, carry):
            k0 = b0 + p * 2
            stage_wait(k0, 0)
            stage_start(jnp.minimum(k0 + 1, b_end - 1), 1)
            stage_pack(k0, 0)
            stage_wait(jnp.minimum(k0 + 1, b_end - 1), 1)
            stage_start(jnp.minimum(k0 + 2, b_end - 1), 0)

            @pl.when(k0 + 1 < b_end)
            def _():
                stage_pack(k0 + 1, 1)
            return carry

        lax.fori_loop(0, (tile_blocks + 1) // 2, stage_pair, 0)
        stage_wait(b_end - 1, 0)
        pltpu.sync_copy(w_hbm, w_v)
        pltpu.sync_copy(b_hbm, b_v)
        plsc.subcore_barrier()

        w_regs = [w_v[pl.ds(j * L, L)] for j in range(2 * DC2)]
        b_reg = b_v[...]
        iota = lax.iota(jnp.int32, L)
        lane_masks = [iota == e for e in range(L)]
        bufs = (uv_buf0, uv_buf1)

        wid = sidx * NC + cidx
        HC = WCHUNKS // 2

        def start(k, slot):
            pltpu.async_copy(h_sp.at[uvidx_v.at[k]], bufs[slot], sems[slot])

        def wait(k, slot):
            pltpu.make_async_copy(
                h_sp.at[uvidx_v.at[k]], bufs[slot], sems[slot]).wait()

        def compute(k, slot):
            buf = bufs[slot]
            for g in range(CE // L):
                e0 = g * L
                accs = [None] * L
                for c in range(DC2):
                    w0 = w_regs[2 * c]
                    w1 = w_regs[2 * c + 1]
                    for e in range(L):
                        u32 = plsc.bitcast(
                            buf[e0 + e, pl.ds(c * L, L)], jnp.bfloat16)
                        v32 = plsc.bitcast(
                            buf[CE + e0 + e, pl.ds(c * L, L)], jnp.bfloat16)
                        p32 = u32 * v32
                        p0, p1 = plsc.unpack(
                            p32, format=plsc.PackFormat.INTERLEAVED)
                        p = p0 * w0 + p1 * w1
                        accs[e] = p if c == 0 else accs[e] + p
                tot = b_reg
                for e in range(L):
                    s = jnp.sum(accs[e])
                    tot = jnp.where(lane_masks[e],
                                    jnp.broadcast_to(s, (L,)), tot)
                out_v[pl.ds(k * CE + e0, L)] = tot + b_reg

        npair = HC // 2

        def half_body(hh, carry):
            half_base = wid * WCHUNKS + hh * HC
            pltpu.sync_copy(uvidx_hbm.at[pl.ds(half_base, HC)], uvidx_v)
            start(0, 0)

            def pair_body(p, carry2):
                k0 = p * 2
                wait(k0, 0)
                start(k0 + 1, 1)
                compute(k0, 0)
                wait(k0 + 1, 1)
                start(jnp.minimum(k0 + 2, HC - 1), 0)
                compute(k0 + 1, 1)
                return carry2

            lax.fori_loop(0, npair, pair_body, 0)
            wait(HC - 1, 0)
            pltpu.sync_copy(out_v, out_hbm.at[pl.ds(half_base * CE, HC * CE)])
            return carry

        lax.fori_loop(0, 2, half_body, 0)

    return sc_kernel


def kernel(h, edge_index, W1_w, W1_b):
    e = edge_index.shape[1]
    e_pad = NW * WCHUNKS * CE
    assert e <= e_pad
    src = edge_index[0].astype(jnp.int32)
    dst = edge_index[1].astype(jnp.int32)
    pad = e_pad - e
    if pad:
        src = jnp.concatenate([src, jnp.zeros((pad,), jnp.int32)])
        dst = jnp.concatenate([dst, jnp.zeros((pad,), jnp.int32)])
    uvidx = jnp.concatenate(
        [src.reshape(-1, CE), dst.reshape(-1, CE)], axis=1)
    w = W1_w.reshape(D).astype(jnp.float32)
    bvec = jnp.broadcast_to(W1_b.reshape(1).astype(jnp.float32), (L,))
    n = h.shape[0]
    out = _make_sc_kernel(e_pad, n)(
        h.astype(jnp.float32), uvidx, w, bvec)
    return out[:e]
